# baseline (speedup 1.0000x reference)
import functools

import jax
import jax.numpy as jnp
from jax import lax
from jax.experimental import pallas as pl
from jax.experimental.pallas import tpu as pltpu
from jax.experimental.pallas import tpu_sc as plsc

NC = 2
NS = 16
NW = NC * NS
LANES = 16

_MESH = dict(core_axis_name="c", subcore_axis_name="s", num_cores=NC,
             num_subcores=NS)


def _wid():
    return lax.axis_index("s") * NC + lax.axis_index("c")


def _make_deg_kernel(n_pad, epc):
    mesh = plsc.VectorSubcoreMesh(**_MESH)
    hrows = n_pad // LANES
    rpt = hrows // NS

    @functools.partial(
        pl.kernel,
        out_type=jax.ShapeDtypeStruct((hrows, LANES), jnp.float32),
        mesh=mesh,
        compiler_params=pltpu.CompilerParams(needs_layout_passes=False),
        scratch_types=[
            pltpu.VMEM((hrows, LANES), jnp.float32),
            pltpu.VMEM((epc,), jnp.int32),
            pltpu.VMEM((5, 128), jnp.int32),
            pltpu.VMEM((rpt, LANES), jnp.float32),
            pltpu.VMEM_SHARED((hrows, LANES), jnp.float32),
        ],
    )
    def deg_kernel(dst_hbm, out_hbm, hist_v, dst_v, idn_v, dinv_v, hist_s):
        cid = lax.axis_index("c")
        sid = lax.axis_index("s")

        @pl.when(cid == 0)
        def _():
            pltpu.sync_copy(dst_hbm.at[sid], dst_v)
            zeros16 = jnp.zeros((LANES,), jnp.float32)
            ones16 = jnp.ones((LANES,), jnp.float32)
            lane = lax.broadcasted_iota(jnp.int32, (LANES,), 0)

            def zero_body(i, c):
                hist_v[i] = zeros16
                return c

            lax.fori_loop(0, hrows, zero_body, 0)
            pltpu.sync_copy(hist_v.at[pl.ds(sid * rpt, rpt)],
                            hist_s.at[pl.ds(sid * rpt, rpt)])

            for r in range(5):
                for j in range(8):
                    idn_v[r, pl.ds(j * LANES, LANES)] = (
                        (r * 8 + j) * LANES + lane)

            def hist_body(i, c):
                idx = dst_v[pl.ds(i * LANES, LANES)]
                plsc.addupdate_scatter(hist_v, [idx >> 4, idx & 15], ones16)
                return c

            lax.fori_loop(0, epc // LANES, hist_body, 0)
            plsc.subcore_barrier()
            for r in range(5):
                pltpu.sync_copy(hist_v.at[pl.ds(r * 128, 128)],
                                hist_s.at[idn_v.at[r]], add=True)
            plsc.subcore_barrier()

            pltpu.sync_copy(hist_s.at[pl.ds(sid * rpt, rpt)], dinv_v)

            def newton(r, c):
                x = dinv_v[r] + 1.0
                yi = 0x5F3759DF - lax.shift_right_logical(
                    plsc.bitcast(x, jnp.int32), 1)
                y = plsc.bitcast(yi, jnp.float32)
                hx = 0.5 * x
                for _ in range(3):
                    y = y * (1.5 - hx * y * y)
                dinv_v[r] = y
                return c

            lax.fori_loop(0, rpt, newton, 0)
            pltpu.sync_copy(dinv_v, out_hbm.at[pl.ds(sid * rpt, rpt)])

    return deg_kernel


def _make_agg_kernel(n_pad, half, nchunk, k):
    mesh = plsc.VectorSubcoreMesh(**_MESH)
    rows_per_tile = n_pad // NS
    zrows = 128

    @functools.partial(
        pl.kernel,
        out_type=jax.ShapeDtypeStruct((NC, n_pad, half), jnp.float32),
        mesh=mesh,
        compiler_params=pltpu.CompilerParams(use_tc_tiling_on_sc=False),
        scratch_types=[
            pltpu.VMEM((nchunk, k), jnp.int32),
            pltpu.VMEM((nchunk, k), jnp.int32),
            pltpu.VMEM((k, half), jnp.float32),
            pltpu.VMEM((k, half), jnp.float32),
            pltpu.VMEM((k, half), jnp.float32),
            pltpu.VMEM((k, half), jnp.float32),
            pltpu.VMEM((zrows, half), jnp.float32),
            pltpu.VMEM_SHARED((n_pad, half), jnp.float32),
            pltpu.SemaphoreType.DMA,
            pltpu.SemaphoreType.DMA,
            pltpu.SemaphoreType.DMA,
            pltpu.SemaphoreType.DMA,
            pltpu.SemaphoreType.DMA,
            pltpu.SemaphoreType.DMA,
            pltpu.SemaphoreType.DMA,
            pltpu.SemaphoreType.DMA,
        ],
    )
    def agg_kernel(ulo_hbm, uhi_hbm, src_hbm, dst_hbm, out_hbm,
                   src_v, dst_v, rows_a, rows_b, rows_c, rows_d, zbuf_v,
                   agg_s, gsem_a, gsem_b, gsem_c, gsem_d,
                   ssem_a, ssem_b, ssem_c, ssem_d):
        cid = lax.axis_index("c")
        sid = lax.axis_index("s")
        pltpu.sync_copy(src_hbm.at[sid], src_v)
        pltpu.sync_copy(dst_hbm.at[sid], dst_v)

        zeros16 = jnp.zeros((LANES,), jnp.float32)

        def zb(i, c):
            for j in range(half // LANES):
                zbuf_v[i, pl.ds(j * LANES, LANES)] = zeros16
            return c

        lax.fori_loop(0, zrows, zb, 0)

        def zcopy(j, c):
            pltpu.sync_copy(
                zbuf_v, agg_s.at[pl.ds(sid * rows_per_tile + j * zrows, zrows)])
            return c

        lax.fori_loop(0, rows_per_tile // zrows, zcopy, 0)
        plsc.subcore_barrier()

        nb = 4
        rows = (rows_a, rows_b, rows_c, rows_d)
        gsem = (gsem_a, gsem_b, gsem_c, gsem_d)
        ssem = (ssem_a, ssem_b, ssem_c, ssem_d)

        def run(u_hbm):
            for b in range(nb):
                pltpu.async_copy(u_hbm.at[src_v.at[b]], rows[b], gsem[b])

            def step(j, c):
                for b in range(nb):
                    i = nb * j + b
                    pltpu.make_async_copy(
                        u_hbm.at[src_v.at[i]], rows[b], gsem[b]).wait()
                    pltpu.async_copy(
                        rows[b], agg_s.at[dst_v.at[i]], ssem[b],
                        add=True).wait()

                    @pl.when(i + nb < nchunk)
                    def _():
                        pltpu.async_copy(
                            u_hbm.at[src_v.at[i + nb]], rows[b], gsem[b])
                return c

            lax.fori_loop(0, nchunk // nb, step, 0)
            for b in range(nchunk % nb):
                i = (nchunk // nb) * nb + b
                pltpu.make_async_copy(
                    u_hbm.at[src_v.at[i]], rows[b], gsem[b]).wait()
                pltpu.sync_copy(rows[b], agg_s.at[dst_v.at[i]], add=True)

        @pl.when(cid == 0)
        def _():
            run(ulo_hbm)

        @pl.when(cid == 1)
        def _():
            run(uhi_hbm)

        plsc.subcore_barrier()
        pltpu.sync_copy(agg_s.at[pl.ds(sid * rows_per_tile, rows_per_tile)],
                        out_hbm.at[cid, pl.ds(sid * rows_per_tile,
                                              rows_per_tile)])

    return agg_kernel


def _make_decode_kernel(hid, nchunk, k):
    mesh = plsc.VectorSubcoreMesh(**_MESH)
    per_tile = nchunk * k

    @functools.partial(
        pl.kernel,
        out_type=jax.ShapeDtypeStruct((NW, per_tile), jnp.float32),
        mesh=mesh,
        compiler_params=pltpu.CompilerParams(needs_layout_passes=False),
        scratch_types=[
            pltpu.VMEM((nchunk, k), jnp.int32),
            pltpu.VMEM((nchunk, k), jnp.int32),
            pltpu.VMEM((k, hid), jnp.float32),
            pltpu.VMEM((k, hid), jnp.float32),
            pltpu.VMEM((k, hid), jnp.float32),
            pltpu.VMEM((k, hid), jnp.float32),
            pltpu.VMEM((per_tile,), jnp.float32),
            pltpu.SemaphoreType.DMA,
            pltpu.SemaphoreType.DMA,
        ],
    )
    def decode_kernel(z_hbm, sl_hbm, dl_hbm, out_hbm,
                      sidx_v, didx_v, srows_a, drows_a, srows_b, drows_b,
                      out_v, sem_a, sem_b):
        wid = _wid()
        pltpu.sync_copy(sl_hbm.at[wid], sidx_v)
        pltpu.sync_copy(dl_hbm.at[wid], didx_v)

        lane = lax.broadcasted_iota(jnp.int32, (LANES,), 0)
        srows = (srows_a, srows_b)
        drows = (drows_a, drows_b)
        sems = (sem_a, sem_b)

        def start(i, b):
            pltpu.async_copy(z_hbm.at[sidx_v.at[i]], srows[b], sems[b])
            pltpu.async_copy(z_hbm.at[didx_v.at[i]], drows[b], sems[b])

        def wait(i, b):
            pltpu.make_async_copy(z_hbm.at[sidx_v.at[i]], srows[b],
                                  sems[b]).wait()
            pltpu.make_async_copy(z_hbm.at[didx_v.at[i]], drows[b],
                                  sems[b]).wait()

        start(0, 0)
        for i in range(nchunk):
            b = i % 2
            if i + 1 < nchunk:
                start(i + 1, 1 - b)
            wait(i, b)

            ng = k // LANES
            accs = tuple(jnp.zeros((LANES,), jnp.float32)
                         for _ in range(ng))

            def feat(d, accs, _b=b):
                dsplat = jnp.broadcast_to(d, (LANES,))
                out = []
                for g in range(ng):
                    rg = lane + g * LANES
                    a = plsc.load_gather(srows[_b], [rg, dsplat])
                    bb = plsc.load_gather(drows[_b], [rg, dsplat])
                    out.append(accs[g] + a * bb)
                return tuple(out)

            accs = lax.fori_loop(0, hid, feat, accs)
            for g in range(ng):
                out_v[pl.ds(i * k + g * LANES, LANES)] = accs[g]
        pltpu.sync_copy(out_v, out_hbm.at[wid])

    return decode_kernel


def _scaled_matmul(x, w, dinv_col, bm, half):
    n, d_in = x.shape
    d_out = w.shape[1]

    def body(x_ref, w_ref, dv_ref, lo_ref, hi_ref):
        u = dv_ref[...] * jnp.dot(
            x_ref[...], w_ref[...], preferred_element_type=jnp.float32)
        lo_ref[...] = u[:, :half]
        hi_ref[...] = u[:, half:]

    return pl.pallas_call(
        body,
        grid=(n // bm,),
        in_specs=[
            pl.BlockSpec((bm, d_in), lambda i: (i, 0)),
            pl.BlockSpec((d_in, d_out), lambda i: (0, 0)),
            pl.BlockSpec((bm, 1), lambda i: (i, 0)),
        ],
        out_specs=[
            pl.BlockSpec((bm, half), lambda i: (i, 0)),
            pl.BlockSpec((bm, half), lambda i: (i, 0)),
        ],
        out_shape=[
            jax.ShapeDtypeStruct((n, half), jnp.float32),
            jax.ShapeDtypeStruct((n, half), jnp.float32),
        ],
    )(x, w, dinv_col)


def _layer2_matmul(aggparts, u1_lo, u1_hi, dinv_col, b1, w2, bm, half):
    n = u1_lo.shape[0]
    hid = 2 * half

    def body(ap_ref, ulo_ref, uhi_ref, dv_ref, b_ref, w_ref, lo_ref, hi_ref):
        zlo = ap_ref[0] + ulo_ref[...]
        zhi = ap_ref[1] + uhi_ref[...]
        dv = dv_ref[...]
        zlo = jnp.maximum(dv * zlo + b_ref[:, :half], 0.0)
        zhi = jnp.maximum(dv * zhi + b_ref[:, half:], 0.0)
        u2 = dv * (
            jnp.dot(zlo, w_ref[:half, :], preferred_element_type=jnp.float32)
            + jnp.dot(zhi, w_ref[half:, :], preferred_element_type=jnp.float32))
        lo_ref[...] = u2[:, :half]
        hi_ref[...] = u2[:, half:]

    return pl.pallas_call(
        body,
        grid=(n // bm,),
        in_specs=[
            pl.BlockSpec((NC, bm, half), lambda i: (0, i, 0)),
            pl.BlockSpec((bm, half), lambda i: (i, 0)),
            pl.BlockSpec((bm, half), lambda i: (i, 0)),
            pl.BlockSpec((bm, 1), lambda i: (i, 0)),
            pl.BlockSpec((1, hid), lambda i: (0, 0)),
            pl.BlockSpec((hid, hid), lambda i: (0, 0)),
        ],
        out_specs=[
            pl.BlockSpec((bm, half), lambda i: (i, 0)),
            pl.BlockSpec((bm, half), lambda i: (i, 0)),
        ],
        out_shape=[
            jax.ShapeDtypeStruct((n, half), jnp.float32),
            jax.ShapeDtypeStruct((n, half), jnp.float32),
        ],
    )(aggparts, u1_lo, u1_hi, dinv_col, b1, w2)


def _final_z(aggparts, u2_lo, u2_hi, dinv_col, b2, bm, half):
    n = u2_lo.shape[0]
    hid = 2 * half

    def body(ap_ref, ulo_ref, uhi_ref, dv_ref, b_ref, o_ref):
        dv = dv_ref[...]
        zlo = dv * (ap_ref[0] + ulo_ref[...]) + b_ref[:, :half]
        zhi = dv * (ap_ref[1] + uhi_ref[...]) + b_ref[:, half:]
        o_ref[...] = jnp.concatenate([zlo, zhi], axis=-1)

    return pl.pallas_call(
        body,
        grid=(n // bm,),
        in_specs=[
            pl.BlockSpec((NC, bm, half), lambda i: (0, i, 0)),
            pl.BlockSpec((bm, half), lambda i: (i, 0)),
            pl.BlockSpec((bm, half), lambda i: (i, 0)),
            pl.BlockSpec((bm, 1), lambda i: (i, 0)),
            pl.BlockSpec((1, hid), lambda i: (0, 0)),
        ],
        out_specs=pl.BlockSpec((bm, hid), lambda i: (i, 0)),
        out_shape=jax.ShapeDtypeStruct((n, hid), jnp.float32),
    )(aggparts, u2_lo, u2_hi, dinv_col, b2)


def kernel(x, edge_index, edge_label_index, W1, b1, W2, b2):
    n, _ = x.shape
    hid = W1.shape[1]
    half = hid // 2
    e = edge_index.shape[1]
    nl = edge_label_index.shape[1]
    bm = 1000

    src = edge_index[0]
    dst = edge_index[1]
    epc = e // NS
    n_pad = -(-n // (NS * 128)) * (NS * 128)

    dinv = _make_deg_kernel(n_pad, epc)(dst.reshape(NS, epc))
    dinv_col = dinv.reshape(n_pad, 1)

    k = 80
    nchunk = epc // k
    src_r = src.reshape(NS, nchunk, k)
    dst_rr = dst.reshape(NS, nchunk, k)
    agg = _make_agg_kernel(n_pad, half, nchunk, k)

    u1_lo, u1_hi = _scaled_matmul(x, W1, dinv_col, bm, half)
    agg1 = agg(u1_lo, u1_hi, src_r, dst_rr)
    u2_lo, u2_hi = _layer2_matmul(agg1, u1_lo, u1_hi, dinv_col,
                                  b1.reshape(1, hid), W2, bm, half)
    agg2 = agg(u2_lo, u2_hi, src_r, dst_rr)
    z2 = _final_z(agg2, u2_lo, u2_hi, dinv_col, b2.reshape(1, hid), bm, half)

    kd = 128
    per_tile = -(-nl // (NW * kd)) * kd
    tot = NW * per_tile
    sl = jnp.pad(edge_label_index[0], (0, tot - nl)).reshape(
        NW, per_tile // kd, kd)
    dl = jnp.pad(edge_label_index[1], (0, tot - nl)).reshape(
        NW, per_tile // kd, kd)
    scores = _make_decode_kernel(hid, per_tile // kd, kd)(z2, sl, dl)
    return scores.reshape(-1)[:nl]


# speedup vs baseline: 1.1680x; 1.1680x over previous
import functools

import jax
import jax.numpy as jnp
from jax import lax
from jax.experimental import pallas as pl
from jax.experimental.pallas import tpu as pltpu
from jax.experimental.pallas import tpu_sc as plsc

NC = 2
NS = 16
NW = NC * NS
LANES = 16

_MESH = dict(core_axis_name="c", subcore_axis_name="s", num_cores=NC,
             num_subcores=NS)


def _wid():
    return lax.axis_index("s") * NC + lax.axis_index("c")


def _make_deg_kernel(n_hist, epw_pad):
    mesh = plsc.VectorSubcoreMesh(**_MESH)

    @functools.partial(
        pl.kernel,
        out_type=jax.ShapeDtypeStruct((NW, n_hist), jnp.float32),
        mesh=mesh,
        compiler_params=pltpu.CompilerParams(needs_layout_passes=False),
        scratch_types=[
            pltpu.VMEM((n_hist,), jnp.float32),
            pltpu.VMEM((epw_pad,), jnp.int32),
        ],
    )
    def deg_kernel(dst_hbm, out_hbm, hist_v, dst_v):
        wid = _wid()
        pltpu.sync_copy(dst_hbm.at[wid], dst_v)
        zeros16 = jnp.zeros((LANES,), jnp.float32)
        ones16 = jnp.ones((LANES,), jnp.float32)

        def zero_body(i, c):
            hist_v[pl.ds(i * LANES, LANES)] = zeros16
            return c

        lax.fori_loop(0, n_hist // LANES, zero_body, 0)

        def hist_body(i, c):
            idx = dst_v[pl.ds(i * LANES, LANES)]
            plsc.addupdate_scatter(hist_v, [idx], ones16)
            return c

        lax.fori_loop(0, epw_pad // LANES, hist_body, 0)
        pltpu.sync_copy(hist_v, out_hbm.at[wid])

    return deg_kernel


def _dinv_tc(degparts):
    nw, n_hist = degparts.shape

    def body(d_ref, o_ref):
        s = jnp.sum(d_ref[...], axis=0, keepdims=True)
        o_ref[...] = lax.rsqrt(s + 1.0)

    return pl.pallas_call(
        body,
        out_shape=jax.ShapeDtypeStruct((1, n_hist), jnp.float32),
    )(degparts)


def _make_agg_kernel(n_pad, half, nchunk, k):
    mesh = plsc.VectorSubcoreMesh(**_MESH)
    rows_per_tile = n_pad // NS
    zrows = 128

    @functools.partial(
        pl.kernel,
        out_type=jax.ShapeDtypeStruct((NC, n_pad, half), jnp.float32),
        mesh=mesh,
        compiler_params=pltpu.CompilerParams(use_tc_tiling_on_sc=False),
        scratch_types=[
            pltpu.VMEM((nchunk, k), jnp.int32),
            pltpu.VMEM((nchunk, k), jnp.int32),
            pltpu.VMEM((k, half), jnp.float32),
            pltpu.VMEM((k, half), jnp.float32),
            pltpu.VMEM((k, half), jnp.float32),
            pltpu.VMEM((k, half), jnp.float32),
            pltpu.VMEM((zrows, half), jnp.float32),
            pltpu.VMEM_SHARED((n_pad, half), jnp.float32),
            pltpu.SemaphoreType.DMA,
            pltpu.SemaphoreType.DMA,
            pltpu.SemaphoreType.DMA,
            pltpu.SemaphoreType.DMA,
            pltpu.SemaphoreType.DMA,
            pltpu.SemaphoreType.DMA,
            pltpu.SemaphoreType.DMA,
            pltpu.SemaphoreType.DMA,
        ],
    )
    def agg_kernel(ulo_hbm, uhi_hbm, src_hbm, dst_hbm, out_hbm,
                   src_v, dst_v, rows_a, rows_b, rows_c, rows_d, zbuf_v,
                   agg_s, gsem_a, gsem_b, gsem_c, gsem_d,
                   ssem_a, ssem_b, ssem_c, ssem_d):
        cid = lax.axis_index("c")
        sid = lax.axis_index("s")
        pltpu.sync_copy(src_hbm.at[sid], src_v)
        pltpu.sync_copy(dst_hbm.at[sid], dst_v)

        zeros16 = jnp.zeros((LANES,), jnp.float32)

        def zb(i, c):
            for j in range(half // LANES):
                zbuf_v[i, pl.ds(j * LANES, LANES)] = zeros16
            return c

        lax.fori_loop(0, zrows, zb, 0)

        def zcopy(j, c):
            pltpu.sync_copy(
                zbuf_v, agg_s.at[pl.ds(sid * rows_per_tile + j * zrows, zrows)])
            return c

        lax.fori_loop(0, rows_per_tile // zrows, zcopy, 0)
        plsc.subcore_barrier()

        nb = 4
        rows = (rows_a, rows_b, rows_c, rows_d)
        gsem = (gsem_a, gsem_b, gsem_c, gsem_d)
        ssem = (ssem_a, ssem_b, ssem_c, ssem_d)

        def run(u_hbm):
            for b in range(nb):
                pltpu.async_copy(u_hbm.at[src_v.at[b]], rows[b], gsem[b])

            def step(j, c):
                for b in range(nb):
                    i = nb * j + b
                    pltpu.make_async_copy(
                        u_hbm.at[src_v.at[i]], rows[b], gsem[b]).wait()
                    pltpu.async_copy(
                        rows[b], agg_s.at[dst_v.at[i]], ssem[b],
                        add=True).wait()

                    @pl.when(i + nb < nchunk)
                    def _():
                        pltpu.async_copy(
                            u_hbm.at[src_v.at[i + nb]], rows[b], gsem[b])
                return c

            lax.fori_loop(0, nchunk // nb, step, 0)
            for b in range(nchunk % nb):
                i = (nchunk // nb) * nb + b
                pltpu.make_async_copy(
                    u_hbm.at[src_v.at[i]], rows[b], gsem[b]).wait()
                pltpu.sync_copy(rows[b], agg_s.at[dst_v.at[i]], add=True)

        @pl.when(cid == 0)
        def _():
            run(ulo_hbm)

        @pl.when(cid == 1)
        def _():
            run(uhi_hbm)

        plsc.subcore_barrier()
        pltpu.sync_copy(agg_s.at[pl.ds(sid * rows_per_tile, rows_per_tile)],
                        out_hbm.at[cid, pl.ds(sid * rows_per_tile,
                                              rows_per_tile)])

    return agg_kernel


def _make_decode_kernel(hid, nchunk, k):
    mesh = plsc.VectorSubcoreMesh(**_MESH)
    per_tile = nchunk * k

    @functools.partial(
        pl.kernel,
        out_type=jax.ShapeDtypeStruct((NW, per_tile), jnp.float32),
        mesh=mesh,
        compiler_params=pltpu.CompilerParams(needs_layout_passes=False),
        scratch_types=[
            pltpu.VMEM((nchunk, k), jnp.int32),
            pltpu.VMEM((nchunk, k), jnp.int32),
            pltpu.VMEM((k, hid), jnp.float32),
            pltpu.VMEM((k, hid), jnp.float32),
            pltpu.VMEM((k, hid), jnp.float32),
            pltpu.VMEM((k, hid), jnp.float32),
            pltpu.VMEM((per_tile,), jnp.float32),
            pltpu.SemaphoreType.DMA,
            pltpu.SemaphoreType.DMA,
        ],
    )
    def decode_kernel(z_hbm, sl_hbm, dl_hbm, out_hbm,
                      sidx_v, didx_v, srows_a, drows_a, srows_b, drows_b,
                      out_v, sem_a, sem_b):
        wid = _wid()
        pltpu.sync_copy(sl_hbm.at[wid], sidx_v)
        pltpu.sync_copy(dl_hbm.at[wid], didx_v)

        lane = lax.broadcasted_iota(jnp.int32, (LANES,), 0)
        srows = (srows_a, srows_b)
        drows = (drows_a, drows_b)
        sems = (sem_a, sem_b)

        def start(i, b):
            pltpu.async_copy(z_hbm.at[sidx_v.at[i]], srows[b], sems[b])
            pltpu.async_copy(z_hbm.at[didx_v.at[i]], drows[b], sems[b])

        def wait(i, b):
            pltpu.make_async_copy(z_hbm.at[sidx_v.at[i]], srows[b],
                                  sems[b]).wait()
            pltpu.make_async_copy(z_hbm.at[didx_v.at[i]], drows[b],
                                  sems[b]).wait()

        start(0, 0)
        for i in range(nchunk):
            b = i % 2
            if i + 1 < nchunk:
                start(i + 1, 1 - b)
            wait(i, b)

            def group(g, c2, _i=i, _b=b):
                def edge(e, res):
                    row = g * LANES + e
                    acc = jnp.zeros((LANES,), jnp.float32)
                    for j in range(hid // LANES):
                        a = srows[_b][row, pl.ds(j * LANES, LANES)]
                        bb = drows[_b][row, pl.ds(j * LANES, LANES)]
                        acc = acc + a * bb
                    s = jnp.sum(acc)
                    return jnp.where(lane == e, s, res)

                res = lax.fori_loop(0, LANES, edge,
                                    jnp.zeros((LANES,), jnp.float32))
                out_v[pl.ds(_i * k + g * LANES, LANES)] = res
                return c2

            lax.fori_loop(0, k // LANES, group, 0)
        pltpu.sync_copy(out_v, out_hbm.at[wid])

    return decode_kernel


def _scaled_matmul(x, w, dinv_col, bm, half):
    n, d_in = x.shape
    d_out = w.shape[1]

    def body(x_ref, w_ref, dv_ref, lo_ref, hi_ref):
        u = dv_ref[...] * jnp.dot(
            x_ref[...], w_ref[...], preferred_element_type=jnp.float32)
        lo_ref[...] = u[:, :half]
        hi_ref[...] = u[:, half:]

    return pl.pallas_call(
        body,
        grid=(n // bm,),
        in_specs=[
            pl.BlockSpec((bm, d_in), lambda i: (i, 0)),
            pl.BlockSpec((d_in, d_out), lambda i: (0, 0)),
            pl.BlockSpec((bm, 1), lambda i: (i, 0)),
        ],
        out_specs=[
            pl.BlockSpec((bm, half), lambda i: (i, 0)),
            pl.BlockSpec((bm, half), lambda i: (i, 0)),
        ],
        out_shape=[
            jax.ShapeDtypeStruct((n, half), jnp.float32),
            jax.ShapeDtypeStruct((n, half), jnp.float32),
        ],
    )(x, w, dinv_col)


def _layer2_matmul(aggparts, u1_lo, u1_hi, dinv_col, b1, w2, bm, half):
    n = u1_lo.shape[0]
    hid = 2 * half

    def body(ap_ref, ulo_ref, uhi_ref, dv_ref, b_ref, w_ref, lo_ref, hi_ref):
        zlo = ap_ref[0] + ulo_ref[...]
        zhi = ap_ref[1] + uhi_ref[...]
        dv = dv_ref[...]
        zlo = jnp.maximum(dv * zlo + b_ref[:, :half], 0.0)
        zhi = jnp.maximum(dv * zhi + b_ref[:, half:], 0.0)
        u2 = dv * (
            jnp.dot(zlo, w_ref[:half, :], preferred_element_type=jnp.float32)
            + jnp.dot(zhi, w_ref[half:, :], preferred_element_type=jnp.float32))
        lo_ref[...] = u2[:, :half]
        hi_ref[...] = u2[:, half:]

    return pl.pallas_call(
        body,
        grid=(n // bm,),
        in_specs=[
            pl.BlockSpec((NC, bm, half), lambda i: (0, i, 0)),
            pl.BlockSpec((bm, half), lambda i: (i, 0)),
            pl.BlockSpec((bm, half), lambda i: (i, 0)),
            pl.BlockSpec((bm, 1), lambda i: (i, 0)),
            pl.BlockSpec((1, hid), lambda i: (0, 0)),
            pl.BlockSpec((hid, hid), lambda i: (0, 0)),
        ],
        out_specs=[
            pl.BlockSpec((bm, half), lambda i: (i, 0)),
            pl.BlockSpec((bm, half), lambda i: (i, 0)),
        ],
        out_shape=[
            jax.ShapeDtypeStruct((n, half), jnp.float32),
            jax.ShapeDtypeStruct((n, half), jnp.float32),
        ],
    )(aggparts, u1_lo, u1_hi, dinv_col, b1, w2)


def _final_z(aggparts, u2_lo, u2_hi, dinv_col, b2, bm, half):
    n = u2_lo.shape[0]
    hid = 2 * half

    def body(ap_ref, ulo_ref, uhi_ref, dv_ref, b_ref, o_ref):
        dv = dv_ref[...]
        zlo = dv * (ap_ref[0] + ulo_ref[...]) + b_ref[:, :half]
        zhi = dv * (ap_ref[1] + uhi_ref[...]) + b_ref[:, half:]
        o_ref[...] = jnp.concatenate([zlo, zhi], axis=-1)

    return pl.pallas_call(
        body,
        grid=(n // bm,),
        in_specs=[
            pl.BlockSpec((NC, bm, half), lambda i: (0, i, 0)),
            pl.BlockSpec((bm, half), lambda i: (i, 0)),
            pl.BlockSpec((bm, half), lambda i: (i, 0)),
            pl.BlockSpec((bm, 1), lambda i: (i, 0)),
            pl.BlockSpec((1, hid), lambda i: (0, 0)),
        ],
        out_specs=pl.BlockSpec((bm, hid), lambda i: (i, 0)),
        out_shape=jax.ShapeDtypeStruct((n, hid), jnp.float32),
    )(aggparts, u2_lo, u2_hi, dinv_col, b2)


def kernel(x, edge_index, edge_label_index, W1, b1, W2, b2):
    n, _ = x.shape
    hid = W1.shape[1]
    half = hid // 2
    e = edge_index.shape[1]
    nl = edge_label_index.shape[1]
    bm = 1000

    src = edge_index[0]
    dst = edge_index[1]
    epc = e // NS
    n_pad = -(-n // (NS * 128)) * (NS * 128)

    epw = e // NW
    pad = (-epw) % LANES
    n_hist = n + LANES
    dst_r = dst.reshape(NW, epw)
    sent = jnp.arange(n, n + pad, dtype=jnp.int32)
    dst_pad = jnp.concatenate(
        [dst_r, jnp.broadcast_to(sent, (NW, pad))], axis=1)
    degparts = _make_deg_kernel(n_hist, epw + pad)(dst_pad)

    dinv_row = _dinv_tc(degparts)
    dinv_col = dinv_row[0, :n].reshape(n, 1)

    k = 80
    nchunk = epc // k
    src_r = src.reshape(NS, nchunk, k)
    dst_rr = dst.reshape(NS, nchunk, k)
    agg = _make_agg_kernel(n_pad, half, nchunk, k)

    u1_lo, u1_hi = _scaled_matmul(x, W1, dinv_col, bm, half)
    agg1 = agg(u1_lo, u1_hi, src_r, dst_rr)
    u2_lo, u2_hi = _layer2_matmul(agg1, u1_lo, u1_hi, dinv_col,
                                  b1.reshape(1, hid), W2, bm, half)
    agg2 = agg(u2_lo, u2_hi, src_r, dst_rr)
    z2 = _final_z(agg2, u2_lo, u2_hi, dinv_col, b2.reshape(1, hid), bm, half)

    kd = 128
    per_tile = -(-nl // (NW * kd)) * kd
    tot = NW * per_tile
    sl = jnp.pad(edge_label_index[0], (0, tot - nl)).reshape(
        NW, per_tile // kd, kd)
    dl = jnp.pad(edge_label_index[1], (0, tot - nl)).reshape(
        NW, per_tile // kd, kd)
    scores = _make_decode_kernel(hid, per_tile // kd, kd)(z2, sl, dl)
    return scores.reshape(-1)[:nl]


# speedup vs baseline: 1.1926x; 1.0210x over previous
import functools

import jax
import jax.numpy as jnp
from jax import lax
from jax.experimental import pallas as pl
from jax.experimental.pallas import tpu as pltpu
from jax.experimental.pallas import tpu_sc as plsc

NC = 2
NS = 16
NW = NC * NS
LANES = 16

_MESH = dict(core_axis_name="c", subcore_axis_name="s", num_cores=NC,
             num_subcores=NS)


def _wid():
    return lax.axis_index("s") * NC + lax.axis_index("c")


def _make_deg_kernel(n_hist, epw_pad):
    mesh = plsc.VectorSubcoreMesh(**_MESH)

    @functools.partial(
        pl.kernel,
        out_type=jax.ShapeDtypeStruct((NW, n_hist), jnp.float32),
        mesh=mesh,
        compiler_params=pltpu.CompilerParams(needs_layout_passes=False),
        scratch_types=[
            pltpu.VMEM((n_hist,), jnp.float32),
            pltpu.VMEM((epw_pad,), jnp.int32),
        ],
    )
    def deg_kernel(dst_hbm, out_hbm, hist_v, dst_v):
        wid = _wid()
        pltpu.sync_copy(dst_hbm.at[wid], dst_v)
        zeros16 = jnp.zeros((LANES,), jnp.float32)
        ones16 = jnp.ones((LANES,), jnp.float32)

        def zero_body(i, c):
            hist_v[pl.ds(i * LANES, LANES)] = zeros16
            return c

        lax.fori_loop(0, n_hist // LANES, zero_body, 0)

        def hist_body(i, c):
            idx = dst_v[pl.ds(i * LANES, LANES)]
            plsc.addupdate_scatter(hist_v, [idx], ones16)
            return c

        lax.fori_loop(0, epw_pad // LANES, hist_body, 0)
        pltpu.sync_copy(hist_v, out_hbm.at[wid])

    return deg_kernel


def _dinv_tc(degparts):
    nw, n_hist = degparts.shape

    def body(d_ref, o_ref):
        s = jnp.sum(d_ref[...], axis=0, keepdims=True)
        o_ref[...] = lax.rsqrt(s + 1.0)

    return pl.pallas_call(
        body,
        out_shape=jax.ShapeDtypeStruct((1, n_hist), jnp.float32),
    )(degparts)


def _make_agg_kernel(n_pad, half, nchunk, k):
    mesh = plsc.VectorSubcoreMesh(**_MESH)
    rows_per_tile = n_pad // NS
    zrows = 128

    @functools.partial(
        pl.kernel,
        out_type=jax.ShapeDtypeStruct((NC, n_pad, half), jnp.float32),
        mesh=mesh,
        compiler_params=pltpu.CompilerParams(use_tc_tiling_on_sc=False),
        scratch_types=[
            pltpu.VMEM((nchunk, k), jnp.int32),
            pltpu.VMEM((nchunk, k), jnp.int32),
            pltpu.VMEM((k, half), jnp.float32),
            pltpu.VMEM((k, half), jnp.float32),
            pltpu.VMEM((k, half), jnp.float32),
            pltpu.VMEM((k, half), jnp.float32),
            pltpu.VMEM((zrows, half), jnp.float32),
            pltpu.VMEM_SHARED((n_pad, half), jnp.float32),
            pltpu.SemaphoreType.DMA,
            pltpu.SemaphoreType.DMA,
            pltpu.SemaphoreType.DMA,
            pltpu.SemaphoreType.DMA,
            pltpu.SemaphoreType.DMA,
            pltpu.SemaphoreType.DMA,
            pltpu.SemaphoreType.DMA,
            pltpu.SemaphoreType.DMA,
        ],
    )
    def agg_kernel(ulo_hbm, uhi_hbm, src_hbm, dst_hbm, out_hbm,
                   src_v, dst_v, rows_a, rows_b, rows_c, rows_d, zbuf_v,
                   agg_s, gsem_a, gsem_b, gsem_c, gsem_d,
                   ssem_a, ssem_b, ssem_c, ssem_d):
        cid = lax.axis_index("c")
        sid = lax.axis_index("s")
        pltpu.sync_copy(src_hbm.at[sid], src_v)
        pltpu.sync_copy(dst_hbm.at[sid], dst_v)

        zeros16 = jnp.zeros((LANES,), jnp.float32)

        def zb(i, c):
            for j in range(half // LANES):
                zbuf_v[i, pl.ds(j * LANES, LANES)] = zeros16
            return c

        lax.fori_loop(0, zrows, zb, 0)

        def zcopy(j, c):
            pltpu.sync_copy(
                zbuf_v, agg_s.at[pl.ds(sid * rows_per_tile + j * zrows, zrows)])
            return c

        lax.fori_loop(0, rows_per_tile // zrows, zcopy, 0)
        plsc.subcore_barrier()

        nb = 4
        rows = (rows_a, rows_b, rows_c, rows_d)
        gsem = (gsem_a, gsem_b, gsem_c, gsem_d)
        ssem = (ssem_a, ssem_b, ssem_c, ssem_d)

        def run(u_hbm):
            for b in range(nb):
                pltpu.async_copy(u_hbm.at[src_v.at[b]], rows[b], gsem[b])

            def step(j, c):
                for b in range(nb):
                    i = nb * j + b
                    pltpu.make_async_copy(
                        u_hbm.at[src_v.at[i]], rows[b], gsem[b]).wait()
                    pltpu.async_copy(
                        rows[b], agg_s.at[dst_v.at[i]], ssem[b],
                        add=True).wait()

                    @pl.when(i + nb < nchunk)
                    def _():
                        pltpu.async_copy(
                            u_hbm.at[src_v.at[i + nb]], rows[b], gsem[b])
                return c

            lax.fori_loop(0, nchunk // nb, step, 0)
            for b in range(nchunk % nb):
                i = (nchunk // nb) * nb + b
                pltpu.make_async_copy(
                    u_hbm.at[src_v.at[i]], rows[b], gsem[b]).wait()
                pltpu.sync_copy(rows[b], agg_s.at[dst_v.at[i]], add=True)

        @pl.when(cid == 0)
        def _():
            run(ulo_hbm)

        @pl.when(cid == 1)
        def _():
            run(uhi_hbm)

        plsc.subcore_barrier()
        pltpu.sync_copy(agg_s.at[pl.ds(sid * rows_per_tile, rows_per_tile)],
                        out_hbm.at[cid, pl.ds(sid * rows_per_tile,
                                              rows_per_tile)])

    return agg_kernel


def _make_decode_kernel(hid, nchunk, k):
    mesh = plsc.VectorSubcoreMesh(**_MESH)
    per_tile = nchunk * k

    @functools.partial(
        pl.kernel,
        out_type=jax.ShapeDtypeStruct((NW, per_tile), jnp.float32),
        mesh=mesh,
        compiler_params=pltpu.CompilerParams(needs_layout_passes=False),
        scratch_types=[
            pltpu.VMEM((nchunk, k), jnp.int32),
            pltpu.VMEM((nchunk, k), jnp.int32),
            pltpu.VMEM((k, hid), jnp.float32),
            pltpu.VMEM((k, hid), jnp.float32),
            pltpu.VMEM((k, hid), jnp.float32),
            pltpu.VMEM((k, hid), jnp.float32),
            pltpu.VMEM((per_tile,), jnp.float32),
            pltpu.SemaphoreType.DMA,
            pltpu.SemaphoreType.DMA,
        ],
    )
    def decode_kernel(z_hbm, sl_hbm, dl_hbm, out_hbm,
                      sidx_v, didx_v, srows_a, drows_a, srows_b, drows_b,
                      out_v, sem_a, sem_b):
        wid = _wid()
        pltpu.sync_copy(sl_hbm.at[wid], sidx_v)
        pltpu.sync_copy(dl_hbm.at[wid], didx_v)

        lane = lax.broadcasted_iota(jnp.int32, (LANES,), 0)
        srows = (srows_a, srows_b)
        drows = (drows_a, drows_b)
        sems = (sem_a, sem_b)

        def start(i, b):
            pltpu.async_copy(z_hbm.at[sidx_v.at[i]], srows[b], sems[b])
            pltpu.async_copy(z_hbm.at[didx_v.at[i]], drows[b], sems[b])

        def wait(i, b):
            pltpu.make_async_copy(z_hbm.at[sidx_v.at[i]], srows[b],
                                  sems[b]).wait()
            pltpu.make_async_copy(z_hbm.at[didx_v.at[i]], drows[b],
                                  sems[b]).wait()

        start(0, 0)
        for i in range(nchunk):
            b = i % 2
            if i + 1 < nchunk:
                start(i + 1, 1 - b)
            wait(i, b)

            def group(g, c2, _i=i, _b=b):
                def edge(e, res):
                    row = g * LANES + e
                    acc = jnp.zeros((LANES,), jnp.float32)
                    for j in range(hid // LANES):
                        a = srows[_b][row, pl.ds(j * LANES, LANES)]
                        bb = drows[_b][row, pl.ds(j * LANES, LANES)]
                        acc = acc + a * bb
                    s = jnp.sum(acc)
                    return jnp.where(lane == e, s, res)

                res = lax.fori_loop(0, LANES, edge,
                                    jnp.zeros((LANES,), jnp.float32))
                out_v[pl.ds(_i * k + g * LANES, LANES)] = res
                return c2

            lax.fori_loop(0, k // LANES, group, 0)
        pltpu.sync_copy(out_v, out_hbm.at[wid])

    return decode_kernel


def _scaled_matmul(x, w, dinv_col, bm, half):
    n, d_in = x.shape
    d_out = w.shape[1]

    def body(x_ref, w_ref, dv_ref, lo_ref, hi_ref):
        u = dv_ref[...] * jnp.dot(
            x_ref[...], w_ref[...], preferred_element_type=jnp.float32)
        lo_ref[...] = u[:, :half]
        hi_ref[...] = u[:, half:]

    return pl.pallas_call(
        body,
        grid=(n // bm,),
        in_specs=[
            pl.BlockSpec((bm, d_in), lambda i: (i, 0)),
            pl.BlockSpec((d_in, d_out), lambda i: (0, 0)),
            pl.BlockSpec((bm, 1), lambda i: (i, 0)),
        ],
        out_specs=[
            pl.BlockSpec((bm, half), lambda i: (i, 0)),
            pl.BlockSpec((bm, half), lambda i: (i, 0)),
        ],
        out_shape=[
            jax.ShapeDtypeStruct((n, half), jnp.float32),
            jax.ShapeDtypeStruct((n, half), jnp.float32),
        ],
    )(x, w, dinv_col)


def _layer2_matmul(aggparts, u1_lo, u1_hi, dinv_col, b1, w2, bm, half):
    n = u1_lo.shape[0]
    hid = 2 * half

    def body(ap_ref, ulo_ref, uhi_ref, dv_ref, b_ref, w_ref, lo_ref, hi_ref):
        zlo = ap_ref[0] + ulo_ref[...]
        zhi = ap_ref[1] + uhi_ref[...]
        dv = dv_ref[...]
        zlo = jnp.maximum(dv * zlo + b_ref[:, :half], 0.0)
        zhi = jnp.maximum(dv * zhi + b_ref[:, half:], 0.0)
        u2 = dv * (
            jnp.dot(zlo, w_ref[:half, :], preferred_element_type=jnp.float32)
            + jnp.dot(zhi, w_ref[half:, :], preferred_element_type=jnp.float32))
        lo_ref[...] = u2[:, :half]
        hi_ref[...] = u2[:, half:]

    return pl.pallas_call(
        body,
        grid=(n // bm,),
        in_specs=[
            pl.BlockSpec((NC, bm, half), lambda i: (0, i, 0)),
            pl.BlockSpec((bm, half), lambda i: (i, 0)),
            pl.BlockSpec((bm, half), lambda i: (i, 0)),
            pl.BlockSpec((bm, 1), lambda i: (i, 0)),
            pl.BlockSpec((1, hid), lambda i: (0, 0)),
            pl.BlockSpec((hid, hid), lambda i: (0, 0)),
        ],
        out_specs=[
            pl.BlockSpec((bm, half), lambda i: (i, 0)),
            pl.BlockSpec((bm, half), lambda i: (i, 0)),
        ],
        out_shape=[
            jax.ShapeDtypeStruct((n, half), jnp.float32),
            jax.ShapeDtypeStruct((n, half), jnp.float32),
        ],
    )(aggparts, u1_lo, u1_hi, dinv_col, b1, w2)


def _final_z(aggparts, u2_lo, u2_hi, dinv_col, b2, bm, half):
    n = u2_lo.shape[0]
    hid = 2 * half

    def body(ap_ref, ulo_ref, uhi_ref, dv_ref, b_ref, o_ref):
        dv = dv_ref[...]
        zlo = dv * (ap_ref[0] + ulo_ref[...]) + b_ref[:, :half]
        zhi = dv * (ap_ref[1] + uhi_ref[...]) + b_ref[:, half:]
        o_ref[...] = jnp.concatenate([zlo, zhi], axis=-1)

    return pl.pallas_call(
        body,
        grid=(n // bm,),
        in_specs=[
            pl.BlockSpec((NC, bm, half), lambda i: (0, i, 0)),
            pl.BlockSpec((bm, half), lambda i: (i, 0)),
            pl.BlockSpec((bm, half), lambda i: (i, 0)),
            pl.BlockSpec((bm, 1), lambda i: (i, 0)),
            pl.BlockSpec((1, hid), lambda i: (0, 0)),
        ],
        out_specs=pl.BlockSpec((bm, hid), lambda i: (i, 0)),
        out_shape=jax.ShapeDtypeStruct((n, hid), jnp.float32),
    )(aggparts, u2_lo, u2_hi, dinv_col, b2)


def kernel(x, edge_index, edge_label_index, W1, b1, W2, b2):
    n, _ = x.shape
    hid = W1.shape[1]
    half = hid // 2
    e = edge_index.shape[1]
    nl = edge_label_index.shape[1]
    bm = 2000

    src = edge_index[0]
    dst = edge_index[1]
    epc = e // NS
    n_pad = -(-n // (NS * 128)) * (NS * 128)

    epw = e // NW
    pad = (-epw) % LANES
    n_hist = n + LANES
    dst_r = dst.reshape(NW, epw)
    sent = jnp.arange(n, n + pad, dtype=jnp.int32)
    dst_pad = jnp.concatenate(
        [dst_r, jnp.broadcast_to(sent, (NW, pad))], axis=1)
    degparts = _make_deg_kernel(n_hist, epw + pad)(dst_pad)

    dinv_row = _dinv_tc(degparts)
    dinv_col = dinv_row[0, :n].reshape(n, 1)

    k = 80
    nchunk = epc // k
    src_r = src.reshape(NS, nchunk, k)
    dst_rr = dst.reshape(NS, nchunk, k)
    agg = _make_agg_kernel(n_pad, half, nchunk, k)

    u1_lo, u1_hi = _scaled_matmul(x, W1, dinv_col, bm, half)
    agg1 = agg(u1_lo, u1_hi, src_r, dst_rr)
    u2_lo, u2_hi = _layer2_matmul(agg1, u1_lo, u1_hi, dinv_col,
                                  b1.reshape(1, hid), W2, bm, half)
    agg2 = agg(u2_lo, u2_hi, src_r, dst_rr)
    z2 = _final_z(agg2, u2_lo, u2_hi, dinv_col, b2.reshape(1, hid), bm, half)

    kd = 128
    per_tile = -(-nl // (NW * kd)) * kd
    tot = NW * per_tile
    sl = jnp.pad(edge_label_index[0], (0, tot - nl)).reshape(
        NW, per_tile // kd, kd)
    dl = jnp.pad(edge_label_index[1], (0, tot - nl)).reshape(
        NW, per_tile // kd, kd)
    scores = _make_decode_kernel(hid, per_tile // kd, kd)(z2, sl, dl)
    return scores.reshape(-1)[:nl]
